# Initial kernel scaffold; baseline (speedup 1.0000x reference)
import jax
import jax.numpy as jnp
from jax.experimental import pallas as pl


def kernel(x, split_index, share_emb_0, share_emb_1, spec_embs):
    raise NotImplementedError("write your pallas kernel here")


# speedup vs baseline: 2.1443x; 2.1443x over previous
import jax
import jax.numpy as jnp
from jax.experimental import pallas as pl
from jax.experimental.pallas import tpu as pltpu

E_DIM = 256
N_LAYERS = 2
SHARE_N_E = 512
SPEC_N_E = 256
CB_N = SHARE_N_E + SPEC_N_E
N_MODALITY = 5
SEG = 4096
N_TOK = SEG * N_MODALITY
BETAS = (0.25, 0.25)

BLK = 512
NB = N_TOK // BLK
BPM = SEG // BLK


def _vq_body(x_ref, cb_ref, xq_ref, res_ref, idx_ref, dist_ref, loss_ref):
    x = x_ref[...]
    jidx = jax.lax.broadcasted_iota(jnp.int32, (BLK, CB_N), 1)
    resid = x
    xq_acc = jnp.zeros_like(x)
    losses = []
    for layer in range(N_LAYERS):
        cb = cb_ref[0, layer]
        cb_sq = jnp.sum(cb * cb, axis=1)
        r_sq = jnp.sum(resid * resid, axis=1, keepdims=True)
        d = (r_sq + cb_sq[None, :]
             - 2.0 * jnp.dot(resid, cb.T, preferred_element_type=jnp.float32))
        dist_ref[:, layer, :] = d
        dmin = jnp.min(d, axis=1, keepdims=True)
        idx = jnp.min(jnp.where(d == dmin, jidx, CB_N), axis=1)
        idx_ref[:, layer] = idx
        onehot = (jidx == idx[:, None]).astype(jnp.float32)
        xq = jax.lax.dot_general(onehot, cb, (((1,), (0,)), ((), ())),
                                 precision=jax.lax.Precision.HIGHEST,
                                 preferred_element_type=jnp.float32)
        t = xq - resid
        losses.append(jnp.sum(t * t))
        xq_st = resid + t
        resid = resid - xq_st
        xq_acc = xq_acc + xq_st
    xq_ref[...] = xq_acc
    res_ref[...] = resid
    loss_ref[0, 0, :] = jnp.stack(losses)


def kernel(x, split_index, share_emb_0, share_emb_1, spec_embs):
    del split_index
    shares = jnp.stack([share_emb_0, share_emb_1])
    shares_b = jnp.broadcast_to(shares[None], (N_MODALITY, N_LAYERS, SHARE_N_E, E_DIM))
    spec_t = jnp.transpose(spec_embs, (1, 0, 2, 3))
    cbs = jnp.concatenate([shares_b, spec_t], axis=2)

    grid = (NB,)
    out_shapes = (
        jax.ShapeDtypeStruct((N_TOK, E_DIM), jnp.float32),
        jax.ShapeDtypeStruct((N_TOK, E_DIM), jnp.float32),
        jax.ShapeDtypeStruct((N_TOK, N_LAYERS), jnp.int32),
        jax.ShapeDtypeStruct((N_TOK, N_LAYERS, CB_N), jnp.float32),
        jax.ShapeDtypeStruct((NB, 1, N_LAYERS), jnp.float32),
    )
    in_specs = [
        pl.BlockSpec((BLK, E_DIM), lambda i: (i, 0)),
        pl.BlockSpec((1, N_LAYERS, CB_N, E_DIM), lambda i: (i // BPM, 0, 0, 0)),
    ]
    out_specs = (
        pl.BlockSpec((BLK, E_DIM), lambda i: (i, 0)),
        pl.BlockSpec((BLK, E_DIM), lambda i: (i, 0)),
        pl.BlockSpec((BLK, N_LAYERS), lambda i: (i, 0)),
        pl.BlockSpec((BLK, N_LAYERS, CB_N), lambda i: (i, 0, 0)),
        pl.BlockSpec((1, 1, N_LAYERS), lambda i: (i, 0, 0)),
    )
    xq, res, idx, dist, loss_parts = pl.pallas_call(
        _vq_body,
        grid=grid,
        in_specs=in_specs,
        out_specs=out_specs,
        out_shape=out_shapes,
        compiler_params=pltpu.CompilerParams(
            dimension_semantics=("arbitrary",),
        ),
    )(x, cbs)

    sums = loss_parts.reshape(N_MODALITY, BPM, N_LAYERS).sum(axis=1)
    scale = jnp.asarray([(1.0 + b) for b in BETAS], jnp.float32) / (SEG * E_DIM)
    q_losses = sums.T * scale[:, None]
    return (xq, res, idx, dist, q_losses)


# speedup vs baseline: 3.2788x; 1.5291x over previous
import jax
import jax.numpy as jnp
from jax.experimental import pallas as pl
from jax.experimental.pallas import tpu as pltpu

E_DIM = 256
N_LAYERS = 2
SHARE_N_E = 512
SPEC_N_E = 256
CB_N = SHARE_N_E + SPEC_N_E
N_MODALITY = 5
SEG = 4096
N_TOK = SEG * N_MODALITY
BETAS = (0.25, 0.25)

BLK = 1024
NB = N_TOK // BLK
BPM = SEG // BLK


def _vq_body(x_ref, cb_ref, cbp_ref, xq_ref, res_ref, idx_ref, dist_ref,
             loss_ref, cbsq_ref):
    @pl.when(pl.program_id(0) % BPM == 0)
    def _compute_cb_norms():
        for layer in range(N_LAYERS):
            cb = cb_ref[0, layer]
            cbsq_ref[layer, :] = jnp.sum(cb * cb, axis=1)

    x = x_ref[...]
    jidx = jax.lax.broadcasted_iota(jnp.int32, (BLK, CB_N), 1)
    resid = x
    xq_acc = jnp.zeros_like(x)
    losses = []
    for layer in range(N_LAYERS):
        cb = cb_ref[0, layer]
        r_sq = jnp.sum(resid * resid, axis=1, keepdims=True)
        d = (r_sq + cbsq_ref[layer, :][None, :]
             - 2.0 * jnp.dot(resid, cb.T, preferred_element_type=jnp.float32))
        dist_ref[:, layer, :] = d
        dmin = jnp.min(d, axis=1, keepdims=True)
        idx = jnp.min(jnp.where(d == dmin, jidx, CB_N), axis=1)
        idx_ref[:, layer] = idx
        onehot = (jidx == idx[:, None]).astype(jnp.bfloat16)
        parts = [jnp.dot(onehot, cbp_ref[0, layer, p],
                         preferred_element_type=jnp.float32)
                 for p in range(3)]
        xq = (parts[0] + parts[1]) + parts[2]
        t = xq - resid
        losses.append(jnp.sum(t * t))
        xq_st = resid + t
        resid = resid - xq_st
        xq_acc = xq_acc + xq_st
    xq_ref[...] = xq_acc
    res_ref[...] = resid
    loss_ref[0, 0, :] = jnp.stack(losses)


def kernel(x, split_index, share_emb_0, share_emb_1, spec_embs):
    del split_index
    shares = jnp.stack([share_emb_0, share_emb_1])
    shares_b = jnp.broadcast_to(shares[None], (N_MODALITY, N_LAYERS, SHARE_N_E, E_DIM))
    spec_t = jnp.transpose(spec_embs, (1, 0, 2, 3))
    cbs = jnp.concatenate([shares_b, spec_t], axis=2)
    p1 = cbs.astype(jnp.bfloat16)
    r1 = cbs - p1.astype(jnp.float32)
    p2 = r1.astype(jnp.bfloat16)
    p3 = (r1 - p2.astype(jnp.float32)).astype(jnp.bfloat16)
    cb_parts = jnp.stack([p1, p2, p3], axis=2)

    grid = (NB,)
    out_shapes = (
        jax.ShapeDtypeStruct((N_TOK, E_DIM), jnp.float32),
        jax.ShapeDtypeStruct((N_TOK, E_DIM), jnp.float32),
        jax.ShapeDtypeStruct((N_TOK, N_LAYERS), jnp.int32),
        jax.ShapeDtypeStruct((N_TOK, N_LAYERS, CB_N), jnp.float32),
        jax.ShapeDtypeStruct((NB, 1, N_LAYERS), jnp.float32),
    )
    in_specs = [
        pl.BlockSpec((BLK, E_DIM), lambda i: (i, 0)),
        pl.BlockSpec((1, N_LAYERS, CB_N, E_DIM), lambda i: (i // BPM, 0, 0, 0)),
        pl.BlockSpec((1, N_LAYERS, 3, CB_N, E_DIM), lambda i: (i // BPM, 0, 0, 0, 0)),
    ]
    out_specs = (
        pl.BlockSpec((BLK, E_DIM), lambda i: (i, 0)),
        pl.BlockSpec((BLK, E_DIM), lambda i: (i, 0)),
        pl.BlockSpec((BLK, N_LAYERS), lambda i: (i, 0)),
        pl.BlockSpec((BLK, N_LAYERS, CB_N), lambda i: (i, 0, 0)),
        pl.BlockSpec((1, 1, N_LAYERS), lambda i: (i, 0, 0)),
    )
    xq, res, idx, dist, loss_parts = pl.pallas_call(
        _vq_body,
        grid=grid,
        in_specs=in_specs,
        out_specs=out_specs,
        out_shape=out_shapes,
        scratch_shapes=[pltpu.VMEM((N_LAYERS, CB_N), jnp.float32)],
        compiler_params=pltpu.CompilerParams(
            dimension_semantics=("arbitrary",),
        ),
    )(x, cbs, cb_parts)

    sums = loss_parts.reshape(N_MODALITY, BPM, N_LAYERS).sum(axis=1)
    scale = jnp.asarray([(1.0 + b) for b in BETAS], jnp.float32) / (SEG * E_DIM)
    q_losses = sums.T * scale[:, None]
    return (xq, res, idx, dist, q_losses)
